# Initial kernel scaffold; baseline (speedup 1.0000x reference)
import functools

import jax
import jax.numpy as jnp
from jax import lax
from jax.experimental import pallas as pl
from jax.experimental.pallas import tpu as pltpu
from jax.experimental.pallas import tpu_sc as plsc

N = 10000
E = 160000
D = 256
H = 128
NC = 2
NS = 16
CH = 80
EP = E // NS
NCHUNK = EP // CH
ROWS_PT = N // NS
ZR = 125
CCH = 40
CEP = E // (NC * NS)
CNCHUNK = CEP // CCH

_mesh = plsc.VectorSubcoreMesh(
    core_axis_name="c", subcore_axis_name="s", num_cores=NC, num_subcores=NS
)


@functools.partial(
    pl.kernel,
    out_type=jax.ShapeDtypeStruct((2 * N, H), jnp.float32),
    mesh=_mesh,
    scratch_types=[
        pltpu.VMEM((CH,), jnp.int32),
        pltpu.VMEM((CH,), jnp.int32),
        pltpu.VMEM((CH, H), jnp.float32),
        pltpu.VMEM((ZR, H), jnp.float32),
        pltpu.VMEM_SHARED((N, H), jnp.float32),
        pltpu.SemaphoreType.DMA,
    ],
)
def _sc_seg_sum(hl_hbm, src_hbm, dst_hbm, out_hbm,
                src_v, dst_v, rows_v, zero_v, acc, gsem):
    c = lax.axis_index("c")
    s = lax.axis_index("s")

    def zbody(i, _):
        r = i // (H // 16)
        k = i % (H // 16)
        zero_v[r, pl.ds(k * 16, 16)] = jnp.zeros((16,), jnp.float32)
        return 0
    lax.fori_loop(0, ZR * (H // 16), zbody, 0)
    for z in range(ROWS_PT // ZR):
        pltpu.sync_copy(zero_v, acc.at[pl.ds(s * ROWS_PT + z * ZR, ZR)])
    plsc.subcore_barrier()

    ebase = s * EP

    def body(i, _):
        off = ebase + i * CH
        pltpu.sync_copy(src_hbm.at[c, pl.ds(off, CH)], src_v)
        pltpu.sync_copy(dst_hbm.at[pl.ds(off, CH)], dst_v)
        pltpu.async_copy(hl_hbm.at[src_v], rows_v, gsem).wait()
        pltpu.sync_copy(rows_v, acc.at[dst_v], add=True)
        return 0
    lax.fori_loop(0, NCHUNK, body, 0)

    plsc.subcore_barrier()
    pltpu.sync_copy(
        acc.at[pl.ds(s * ROWS_PT, ROWS_PT)],
        out_hbm.at[pl.ds(c * N + s * ROWS_PT, ROWS_PT)],
    )


@functools.partial(
    pl.kernel,
    out_type=jax.ShapeDtypeStruct((NC, N, 16), jnp.float32),
    mesh=_mesh,
    scratch_types=[
        pltpu.VMEM((CCH,), jnp.int32),
        pltpu.VMEM((CCH, 16), jnp.float32),
        pltpu.VMEM((ZR, 16), jnp.float32),
        pltpu.VMEM_SHARED((N, 16), jnp.float32),
    ],
)
def _sc_counts(dst_hbm, out_hbm, dst_v, ones_v, zero_v, cacc):
    c = lax.axis_index("c")
    s = lax.axis_index("s")

    def fill(i, _):
        zero_v[i, :] = jnp.zeros((16,), jnp.float32)
        return 0
    lax.fori_loop(0, ZR, fill, 0)

    def fill1(i, _):
        ones_v[i, :] = jnp.ones((16,), jnp.float32)
        return 0
    lax.fori_loop(0, CCH, fill1, 0)

    for z in range(ROWS_PT // ZR):
        pltpu.sync_copy(zero_v, cacc.at[pl.ds(s * ROWS_PT + z * ZR, ZR)])
    plsc.subcore_barrier()

    ebase = (c * NS + s) * CEP

    def body(i, _):
        pltpu.sync_copy(dst_hbm.at[pl.ds(ebase + i * CCH, CCH)], dst_v)
        pltpu.sync_copy(ones_v, cacc.at[dst_v], add=True)
        return 0
    lax.fori_loop(0, CNCHUNK, body, 0)

    plsc.subcore_barrier()
    pltpu.sync_copy(
        cacc.at[pl.ds(s * ROWS_PT, ROWS_PT)],
        out_hbm.at[c, pl.ds(s * ROWS_PT, ROWS_PT)],
    )


R = 1000
NB = N // R
KSPLIT = 4


def _tc_entry_body(x_ref, w_ref, z_ref):
    z_ref[...] = jnp.dot(x_ref[...], w_ref[...],
                         preferred_element_type=jnp.float32)


def _tc_entry(x, wall):
    return pl.pallas_call(
        _tc_entry_body,
        grid=(NB, KSPLIT),
        in_specs=[
            pl.BlockSpec((R, D), lambda i, k: (i, 0)),
            pl.BlockSpec((D, H), lambda i, k: (0, k)),
        ],
        out_specs=pl.BlockSpec((R, H), lambda i, k: (k * (N // R) + i, 0)),
        out_shape=jax.ShapeDtypeStruct((4 * N, H), jnp.float32),
    )(x, wall)


def _tc_mid_body(sA_ref, sB_ref, hA_ref, hB_ref, c_ref, b_ref, w_ref, z_ref):
    cnt = c_ref[0][:, 0:1] + c_ref[1][:, 0:1]
    inv = 1.0 / jnp.maximum(cnt, 1.0)
    b = b_ref[...]
    h0 = sA_ref[...] * inv + hA_ref[...] + b[:, :H]
    h1 = sB_ref[...] * inv + hB_ref[...] + b[:, H:]
    h = jax.nn.relu(jnp.concatenate([h0, h1], axis=1))
    z_ref[...] = jnp.dot(h, w_ref[...], preferred_element_type=jnp.float32)


def _tc_mid(s_st, z_prev, cnt_p, b2, wall):
    nbr = N // R
    return pl.pallas_call(
        _tc_mid_body,
        grid=(NB, KSPLIT),
        in_specs=[
            pl.BlockSpec((R, H), lambda i, k: (i, 0)),
            pl.BlockSpec((R, H), lambda i, k: (nbr + i, 0)),
            pl.BlockSpec((R, H), lambda i, k: (2 * nbr + i, 0)),
            pl.BlockSpec((R, H), lambda i, k: (3 * nbr + i, 0)),
            pl.BlockSpec((NC, R, 16), lambda i, k: (0, i, 0)),
            pl.BlockSpec((1, D), lambda i, k: (0, 0)),
            pl.BlockSpec((D, H), lambda i, k: (0, k)),
        ],
        out_specs=pl.BlockSpec((R, H), lambda i, k: (k * nbr + i, 0)),
        out_shape=jax.ShapeDtypeStruct((4 * N, H), jnp.float32),
    )(s_st, s_st, z_prev, z_prev, cnt_p, b2, wall)


def _tc_final_body(sA_ref, sB_ref, hA_ref, hB_ref, c_ref, b_ref, o_ref):
    cnt = c_ref[0][:, 0:1] + c_ref[1][:, 0:1]
    inv = 1.0 / jnp.maximum(cnt, 1.0)
    b = b_ref[...]
    o_ref[:, :H] = sA_ref[...] * inv + hA_ref[...] + b[:, :H]
    o_ref[:, H:] = sB_ref[...] * inv + hB_ref[...] + b[:, H:]


def _tc_final(s_st, z_prev, cnt_p, b2):
    nbr = N // R
    return pl.pallas_call(
        _tc_final_body,
        grid=(NB,),
        in_specs=[
            pl.BlockSpec((R, H), lambda i: (i, 0)),
            pl.BlockSpec((R, H), lambda i: (nbr + i, 0)),
            pl.BlockSpec((R, H), lambda i: (2 * nbr + i, 0)),
            pl.BlockSpec((R, H), lambda i: (3 * nbr + i, 0)),
            pl.BlockSpec((NC, R, 16), lambda i: (0, i, 0)),
            pl.BlockSpec((1, D), lambda i: (0, 0)),
        ],
        out_specs=pl.BlockSpec((R, D), lambda i: (i, 0)),
        out_shape=jax.ShapeDtypeStruct((N, D), jnp.float32),
    )(s_st, s_st, z_prev, z_prev, cnt_p, b2)


def kernel(x, edge_index, edge_attr, Wl0, bl0, Wr0, Wl1, bl1, Wr1, Wl2, bl2, Wr2):
    src = edge_index[0]
    dst = edge_index[1]
    src_st = jnp.stack([src, src + N])

    w0 = jnp.concatenate([Wl0.T, Wr0.T], axis=1)
    w1 = jnp.concatenate([Wl1.T, Wr1.T], axis=1)
    w2 = jnp.concatenate([Wl2.T, Wr2.T], axis=1)
    b0 = bl0.reshape(1, D)
    b1 = bl1.reshape(1, D)
    b2 = bl2.reshape(1, D)

    cnt_p = _sc_counts(dst)

    z0 = _tc_entry(x, w0)
    s0 = _sc_seg_sum(z0[: 2 * N], src_st, dst)
    z1 = _tc_mid(s0, z0, cnt_p, b0, w1)
    s1 = _sc_seg_sum(z1[: 2 * N], src_st, dst)
    z2 = _tc_mid(s1, z1, cnt_p, b1, w2)
    s2 = _sc_seg_sum(z2[: 2 * N], src_st, dst)
    return _tc_final(s2, z2, cnt_p, b2)


# speedup vs baseline: 3.0742x; 3.0742x over previous
import functools

import jax
import jax.numpy as jnp
from jax import lax
from jax.experimental import pallas as pl
from jax.experimental.pallas import tpu as pltpu
from jax.experimental.pallas import tpu_sc as plsc

N = 10000
E = 160000
D = 256
H = 128
NC = 2
NS = 16
CH = 80
EP = E // NS
NCHUNK = EP // CH
ZR = 40
NRCH = N // ZR

@functools.lru_cache(maxsize=None)
def _build_sc_kernels():
    mesh = plsc.VectorSubcoreMesh(
        core_axis_name="c", subcore_axis_name="s",
        num_cores=NC, num_subcores=NS,
    )
    seg = functools.partial(
        pl.kernel,
        out_type=jax.ShapeDtypeStruct((2 * N, H), jnp.float32),
        mesh=mesh,
        scratch_types=[
            pltpu.VMEM((CH,), jnp.int32),
            pltpu.VMEM((CH,), jnp.int32),
            pltpu.VMEM((CH, H), jnp.float32),
            pltpu.VMEM((ZR, H), jnp.float32),
            pltpu.VMEM_SHARED((N, H), jnp.float32),
            pltpu.SemaphoreType.DMA,
        ],
    )(_sc_seg_sum_body)
    cnts = functools.partial(
        pl.kernel,
        out_type=jax.ShapeDtypeStruct((N, H), jnp.float32),
        mesh=mesh,
        scratch_types=[
            pltpu.VMEM((CH,), jnp.int32),
            pltpu.VMEM((CH, H), jnp.float32),
            pltpu.VMEM((ZR, H), jnp.float32),
            pltpu.VMEM_SHARED((N, H), jnp.float32),
        ],
    )(_sc_counts_body)
    return seg, cnts


def _sc_seg_sum_body(hl_hbm, src_hbm, dst_hbm, out_hbm,
                     src_v, dst_v, rows_v, zero_v, acc, gsem):
    c = lax.axis_index("c")
    s = lax.axis_index("s")

    def zbody(i, _):
        r = i // (H // 16)
        k = i % (H // 16)
        zero_v[r, pl.ds(k * 16, 16)] = jnp.zeros((16,), jnp.float32)
        return 0
    lax.fori_loop(0, ZR * (H // 16), zbody, 0)

    rlo = (s * NRCH) // NS
    rhi = ((s + 1) * NRCH) // NS

    def zcopy(j, _):
        pltpu.sync_copy(zero_v, acc.at[pl.ds(j * ZR, ZR)])
        return 0
    lax.fori_loop(rlo, rhi, zcopy, 0)
    plsc.subcore_barrier()

    ebase = s * EP

    def body(i, _):
        off = ebase + i * CH
        pltpu.sync_copy(src_hbm.at[pl.ds(c * E + off, CH)], src_v)
        pltpu.sync_copy(dst_hbm.at[pl.ds(off, CH)], dst_v)
        pltpu.async_copy(hl_hbm.at[src_v], rows_v, gsem).wait()
        pltpu.sync_copy(rows_v, acc.at[dst_v], add=True)
        return 0
    lax.fori_loop(0, NCHUNK, body, 0)

    plsc.subcore_barrier()

    def wcopy(j, _):
        pltpu.sync_copy(acc.at[pl.ds(j * ZR, ZR)],
                        out_hbm.at[pl.ds(c * N + j * ZR, ZR)])
        return 0
    lax.fori_loop(rlo, rhi, wcopy, 0)


def _sc_counts_body(dst_hbm, out_hbm, dst_v, ones_v, zero_v, cacc):
    c = lax.axis_index("c")
    s = lax.axis_index("s")

    def zbody(i, _):
        r = i // (H // 16)
        k = i % (H // 16)
        zero_v[r, pl.ds(k * 16, 16)] = jnp.zeros((16,), jnp.float32)
        return 0
    lax.fori_loop(0, ZR * (H // 16), zbody, 0)

    def obody(i, _):
        r = i // (H // 16)
        k = i % (H // 16)
        ones_v[r, pl.ds(k * 16, 16)] = jnp.ones((16,), jnp.float32)
        return 0
    lax.fori_loop(0, CH * (H // 16), obody, 0)

    rlo = (s * NRCH) // NS
    rhi = ((s + 1) * NRCH) // NS

    def zcopy(j, _):
        pltpu.sync_copy(zero_v, cacc.at[pl.ds(j * ZR, ZR)])
        return 0
    lax.fori_loop(rlo, rhi, zcopy, 0)
    plsc.subcore_barrier()

    ebase = s * EP

    def body(i, _):
        pltpu.sync_copy(dst_hbm.at[pl.ds(ebase + i * CH, CH)], dst_v)
        pltpu.sync_copy(ones_v, cacc.at[dst_v], add=True)
        return 0
    lax.fori_loop(0, NCHUNK, body, 0)

    plsc.subcore_barrier()

    @pl.when(c == 0)
    def _():
        def wcopy(j, _):
            pltpu.sync_copy(cacc.at[pl.ds(j * ZR, ZR)],
                            out_hbm.at[pl.ds(j * ZR, ZR)])
            return 0
        lax.fori_loop(rlo, rhi, wcopy, 0)


R = 1000
NB = N // R
KSPLIT = 4


def _tc_entry_body(x_ref, w_ref, z_ref):
    z_ref[...] = jnp.dot(x_ref[...], w_ref[...],
                         preferred_element_type=jnp.float32)


def _tc_entry(x, wall):
    return pl.pallas_call(
        _tc_entry_body,
        grid=(NB, KSPLIT),
        in_specs=[
            pl.BlockSpec((R, D), lambda i, k: (i, 0)),
            pl.BlockSpec((D, H), lambda i, k: (0, k)),
        ],
        out_specs=pl.BlockSpec((R, H), lambda i, k: (k * (N // R) + i, 0)),
        out_shape=jax.ShapeDtypeStruct((4 * N, H), jnp.float32),
    )(x, wall)


def _tc_mid_body(sA_ref, sB_ref, hA_ref, hB_ref, c_ref, b_ref,
                 w_ref, z_ref):
    inv = 1.0 / jnp.maximum(c_ref[:, 0:1], 1.0)
    b = b_ref[...]
    h0 = sA_ref[...] * inv + hA_ref[...] + b[:, :H]
    h1 = sB_ref[...] * inv + hB_ref[...] + b[:, H:]
    h = jax.nn.relu(jnp.concatenate([h0, h1], axis=1))
    z_ref[...] = jnp.dot(h, w_ref[...], preferred_element_type=jnp.float32)


def _tc_mid(s_st, z_prev, cnt_p, b2, wall):
    nbr = N // R
    return pl.pallas_call(
        _tc_mid_body,
        grid=(NB, KSPLIT),
        in_specs=[
            pl.BlockSpec((R, H), lambda i, k: (i, 0)),
            pl.BlockSpec((R, H), lambda i, k: (nbr + i, 0)),
            pl.BlockSpec((R, H), lambda i, k: (2 * nbr + i, 0)),
            pl.BlockSpec((R, H), lambda i, k: (3 * nbr + i, 0)),
            pl.BlockSpec((R, H), lambda i, k: (i, 0)),
            pl.BlockSpec((1, D), lambda i, k: (0, 0)),
            pl.BlockSpec((D, H), lambda i, k: (0, k)),
        ],
        out_specs=pl.BlockSpec((R, H), lambda i, k: (k * nbr + i, 0)),
        out_shape=jax.ShapeDtypeStruct((4 * N, H), jnp.float32),
    )(s_st, s_st, z_prev, z_prev, cnt_p, b2, wall)


def _tc_final_body(sA_ref, sB_ref, hA_ref, hB_ref, c_ref, b_ref,
                   o_ref):
    inv = 1.0 / jnp.maximum(c_ref[:, 0:1], 1.0)
    b = b_ref[...]
    o_ref[:, :H] = sA_ref[...] * inv + hA_ref[...] + b[:, :H]
    o_ref[:, H:] = sB_ref[...] * inv + hB_ref[...] + b[:, H:]


def _tc_final(s_st, z_prev, cnt_p, b2):
    nbr = N // R
    return pl.pallas_call(
        _tc_final_body,
        grid=(NB,),
        in_specs=[
            pl.BlockSpec((R, H), lambda i: (i, 0)),
            pl.BlockSpec((R, H), lambda i: (nbr + i, 0)),
            pl.BlockSpec((R, H), lambda i: (2 * nbr + i, 0)),
            pl.BlockSpec((R, H), lambda i: (3 * nbr + i, 0)),
            pl.BlockSpec((R, H), lambda i: (i, 0)),
            pl.BlockSpec((1, D), lambda i: (0, 0)),
        ],
        out_specs=pl.BlockSpec((R, D), lambda i: (i, 0)),
        out_shape=jax.ShapeDtypeStruct((N, D), jnp.float32),
    )(s_st, s_st, z_prev, z_prev, cnt_p, b2)


def kernel(x, edge_index, edge_attr, Wl0, bl0, Wr0, Wl1, bl1, Wr1, Wl2, bl2, Wr2):
    src = edge_index[0]
    dst = edge_index[1]
    src_st = jnp.concatenate([src, src + N])

    w0 = jnp.concatenate([Wl0.T, Wr0.T], axis=1)
    w1 = jnp.concatenate([Wl1.T, Wr1.T], axis=1)
    w2 = jnp.concatenate([Wl2.T, Wr2.T], axis=1)
    b0 = bl0.reshape(1, D)
    b1 = bl1.reshape(1, D)
    b2 = bl2.reshape(1, D)

    _sc_seg_sum, _sc_counts = _build_sc_kernels()
    cnt_p = _sc_counts(dst)

    z0 = _tc_entry(x, w0)
    s0 = _sc_seg_sum(z0, src_st, dst)
    z1 = _tc_mid(s0, z0, cnt_p, b0, w1)
    s1 = _sc_seg_sum(z1, src_st, dst)
    z2 = _tc_mid(s1, z1, cnt_p, b1, w2)
    s2 = _sc_seg_sum(z2, src_st, dst)
    return _tc_final(s2, z2, cnt_p, b2)


# speedup vs baseline: 5.2916x; 1.7213x over previous
import functools

import jax
import jax.numpy as jnp
from jax import lax
from jax.experimental import pallas as pl
from jax.experimental.pallas import tpu as pltpu
from jax.experimental.pallas import tpu_sc as plsc

N = 10000
E = 160000
D = 256
H = 128
NC = 2
NS = 16
CH = 80
EP = E // NS
NCHUNK = EP // CH
ZR = 40
NRCH = N // ZR
CCH = 40
CEP = E // (NC * NS)
CNCHUNK = CEP // CCH

@functools.lru_cache(maxsize=None)
def _build_sc_kernels():
    mesh = plsc.VectorSubcoreMesh(
        core_axis_name="c", subcore_axis_name="s",
        num_cores=NC, num_subcores=NS,
    )
    seg = functools.partial(
        pl.kernel,
        out_type=jax.ShapeDtypeStruct((2 * N, H), jnp.float32),
        mesh=mesh,
        scratch_types=[
            pltpu.VMEM((EP,), jnp.int32),
            pltpu.VMEM((CH,), jnp.int32),
            pltpu.VMEM((CH,), jnp.int32),
            pltpu.VMEM((CH, H), jnp.float32),
            pltpu.VMEM((CH, H), jnp.float32),
            pltpu.VMEM((ZR, H), jnp.float32),
            pltpu.VMEM_SHARED((N, H), jnp.float32),
            pltpu.SemaphoreType.DMA,
            pltpu.SemaphoreType.DMA,
            pltpu.SemaphoreType.DMA,
            pltpu.SemaphoreType.DMA,
            pltpu.SemaphoreType.DMA,
        ],
    )(_sc_seg_sum_body)
    cnts = functools.partial(
        pl.kernel,
        out_type=jax.ShapeDtypeStruct((NC * N, H), jnp.float32),
        mesh=mesh,
        scratch_types=[
            pltpu.VMEM((CCH,), jnp.int32),
            pltpu.VMEM((CCH,), jnp.int32),
            pltpu.VMEM((CCH, H), jnp.float32),
            pltpu.VMEM((ZR, H), jnp.float32),
            pltpu.VMEM_SHARED((N, H), jnp.float32),
            pltpu.SemaphoreType.DMA,
            pltpu.SemaphoreType.DMA,
            pltpu.SemaphoreType.DMA,
            pltpu.SemaphoreType.DMA,
        ],
    )(_sc_counts_body)
    return seg, cnts


def _sc_seg_sum_body(hl_hbm, src_hbm, dst_hbm, out_hbm,
                     src_a, dst0, dst1, rows0, rows1, zero_v, acc,
                     gsem, dsem0, dsem1, ssem0, ssem1):
    c = lax.axis_index("c")
    s = lax.axis_index("s")
    rows = (rows0, rows1)
    dsts = (dst0, dst1)
    dsems = (dsem0, dsem1)
    ssems = (ssem0, ssem1)

    pltpu.sync_copy(src_hbm.at[pl.ds(c * E + s * EP, EP)], src_a)

    def zbody(i, _):
        r = i // (H // 16)
        k = i % (H // 16)
        zero_v[r, pl.ds(k * 16, 16)] = jnp.zeros((16,), jnp.float32)
        return 0
    lax.fori_loop(0, ZR * (H // 16), zbody, 0)

    rlo = (s * NRCH) // NS
    rhi = ((s + 1) * NRCH) // NS

    def zcopy(j, _):
        pltpu.sync_copy(zero_v, acc.at[pl.ds(j * ZR, ZR)])
        return 0
    lax.fori_loop(rlo, rhi, zcopy, 0)
    plsc.subcore_barrier()

    def src_sl(j):
        return src_a.at[pl.ds(j * CH, CH)]

    def dst_sl(j):
        return dst_hbm.at[pl.ds(s * EP + j * CH, CH)]

    pltpu.async_copy(dst_sl(0), dsts[0], dsems[0])
    pltpu.async_copy(hl_hbm.at[src_sl(0)], rows[0], gsem)

    def step(j, b, first):
        pltpu.make_async_copy(hl_hbm.at[src_sl(j)], rows[b], gsem).wait()
        pltpu.make_async_copy(dst_sl(j), dsts[b], dsems[b]).wait()
        pltpu.async_copy(rows[b], acc.at[dsts[b]], ssems[b], add=True)
        if not first:
            pltpu.make_async_copy(
                rows[1 - b], acc.at[dsts[1 - b]], ssems[1 - b]).wait()

        @pl.when(j + 1 < NCHUNK)
        def _():
            pltpu.async_copy(dst_sl(j + 1), dsts[1 - b], dsems[1 - b])
            pltpu.async_copy(hl_hbm.at[src_sl(j + 1)], rows[1 - b], gsem)

    step(0, 0, first=True)

    def body(i, _):
        step(1 + 2 * i, 1, first=False)
        step(2 + 2 * i, 0, first=False)
        return 0
    lax.fori_loop(0, (NCHUNK - 1) // 2, body, 0)

    lastb = (NCHUNK - 1) % 2
    pltpu.make_async_copy(
        rows[lastb], acc.at[dsts[lastb]], ssems[lastb]).wait()

    plsc.subcore_barrier()

    def wcopy(j, _):
        pltpu.sync_copy(acc.at[pl.ds(j * ZR, ZR)],
                        out_hbm.at[pl.ds(c * N + j * ZR, ZR)])
        return 0
    lax.fori_loop(rlo, rhi, wcopy, 0)


def _sc_counts_body(dst_hbm, out_hbm, dst0, dst1, ones_v, zero_v, cacc,
                    dsem0, dsem1, ssem0, ssem1):
    c = lax.axis_index("c")
    s = lax.axis_index("s")
    dsts = (dst0, dst1)
    dsems = (dsem0, dsem1)
    ssems = (ssem0, ssem1)

    def zbody(i, _):
        r = i // (H // 16)
        k = i % (H // 16)
        zero_v[r, pl.ds(k * 16, 16)] = jnp.zeros((16,), jnp.float32)
        return 0
    lax.fori_loop(0, ZR * (H // 16), zbody, 0)

    def obody(i, _):
        r = i // (H // 16)
        k = i % (H // 16)
        ones_v[r, pl.ds(k * 16, 16)] = jnp.ones((16,), jnp.float32)
        return 0
    lax.fori_loop(0, CCH * (H // 16), obody, 0)

    rlo = (s * NRCH) // NS
    rhi = ((s + 1) * NRCH) // NS

    def zcopy(j, _):
        pltpu.sync_copy(zero_v, cacc.at[pl.ds(j * ZR, ZR)])
        return 0
    lax.fori_loop(rlo, rhi, zcopy, 0)
    plsc.subcore_barrier()

    ebase = (c * NS + s) * CEP

    def dst_sl(j):
        return dst_hbm.at[pl.ds(ebase + j * CCH, CCH)]

    pltpu.async_copy(dst_sl(0), dsts[0], dsems[0])

    def step(j, b, first):
        pltpu.make_async_copy(dst_sl(j), dsts[b], dsems[b]).wait()
        pltpu.async_copy(ones_v, cacc.at[dsts[b]], ssems[b], add=True)
        if not first:
            pltpu.make_async_copy(
                ones_v, cacc.at[dsts[1 - b]], ssems[1 - b]).wait()

        @pl.when(j + 1 < CNCHUNK)
        def _():
            pltpu.async_copy(dst_sl(j + 1), dsts[1 - b], dsems[1 - b])

    step(0, 0, first=True)

    def body(i, _):
        step(1 + 2 * i, 1, first=False)
        step(2 + 2 * i, 0, first=False)
        return 0
    lax.fori_loop(0, (CNCHUNK - 1) // 2, body, 0)

    lastb = (CNCHUNK - 1) % 2
    pltpu.make_async_copy(
        ones_v, cacc.at[dsts[lastb]], ssems[lastb]).wait()

    plsc.subcore_barrier()

    def wcopy(j, _):
        pltpu.sync_copy(cacc.at[pl.ds(j * ZR, ZR)],
                        out_hbm.at[pl.ds(c * N + j * ZR, ZR)])
        return 0
    lax.fori_loop(rlo, rhi, wcopy, 0)


R = 1000
NB = N // R
KSPLIT = 4


def _tc_entry_body(x_ref, w_ref, z_ref):
    z_ref[...] = jnp.dot(x_ref[...], w_ref[...],
                         preferred_element_type=jnp.float32)


def _tc_entry(x, wall):
    return pl.pallas_call(
        _tc_entry_body,
        grid=(NB, KSPLIT),
        in_specs=[
            pl.BlockSpec((R, D), lambda i, k: (i, 0)),
            pl.BlockSpec((D, H), lambda i, k: (0, k)),
        ],
        out_specs=pl.BlockSpec((R, H), lambda i, k: (k * (N // R) + i, 0)),
        out_shape=jax.ShapeDtypeStruct((4 * N, H), jnp.float32),
    )(x, wall)


def _tc_mid_body(sA_ref, sB_ref, hA_ref, hB_ref, cA_ref, cB_ref, b_ref,
                 w_ref, z_ref):
    cnt = cA_ref[:, 0:1] + cB_ref[:, 0:1]
    inv = 1.0 / jnp.maximum(cnt, 1.0)
    b = b_ref[...]
    h0 = sA_ref[...] * inv + hA_ref[...] + b[:, :H]
    h1 = sB_ref[...] * inv + hB_ref[...] + b[:, H:]
    h = jax.nn.relu(jnp.concatenate([h0, h1], axis=1))
    z_ref[...] = jnp.dot(h, w_ref[...], preferred_element_type=jnp.float32)


def _tc_mid(s_st, z_prev, cnt_p, b2, wall):
    nbr = N // R
    return pl.pallas_call(
        _tc_mid_body,
        grid=(NB, KSPLIT),
        in_specs=[
            pl.BlockSpec((R, H), lambda i, k: (i, 0)),
            pl.BlockSpec((R, H), lambda i, k: (nbr + i, 0)),
            pl.BlockSpec((R, H), lambda i, k: (2 * nbr + i, 0)),
            pl.BlockSpec((R, H), lambda i, k: (3 * nbr + i, 0)),
            pl.BlockSpec((R, H), lambda i, k: (i, 0)),
            pl.BlockSpec((R, H), lambda i, k: (nbr + i, 0)),
            pl.BlockSpec((1, D), lambda i, k: (0, 0)),
            pl.BlockSpec((D, H), lambda i, k: (0, k)),
        ],
        out_specs=pl.BlockSpec((R, H), lambda i, k: (k * nbr + i, 0)),
        out_shape=jax.ShapeDtypeStruct((4 * N, H), jnp.float32),
    )(s_st, s_st, z_prev, z_prev, cnt_p, cnt_p, b2, wall)


def _tc_final_body(sA_ref, sB_ref, hA_ref, hB_ref, cA_ref, cB_ref, b_ref,
                   o_ref):
    cnt = cA_ref[:, 0:1] + cB_ref[:, 0:1]
    inv = 1.0 / jnp.maximum(cnt, 1.0)
    b = b_ref[...]
    o_ref[:, :H] = sA_ref[...] * inv + hA_ref[...] + b[:, :H]
    o_ref[:, H:] = sB_ref[...] * inv + hB_ref[...] + b[:, H:]


def _tc_final(s_st, z_prev, cnt_p, b2):
    nbr = N // R
    return pl.pallas_call(
        _tc_final_body,
        grid=(NB,),
        in_specs=[
            pl.BlockSpec((R, H), lambda i: (i, 0)),
            pl.BlockSpec((R, H), lambda i: (nbr + i, 0)),
            pl.BlockSpec((R, H), lambda i: (2 * nbr + i, 0)),
            pl.BlockSpec((R, H), lambda i: (3 * nbr + i, 0)),
            pl.BlockSpec((R, H), lambda i: (i, 0)),
            pl.BlockSpec((R, H), lambda i: (nbr + i, 0)),
            pl.BlockSpec((1, D), lambda i: (0, 0)),
        ],
        out_specs=pl.BlockSpec((R, D), lambda i: (i, 0)),
        out_shape=jax.ShapeDtypeStruct((N, D), jnp.float32),
    )(s_st, s_st, z_prev, z_prev, cnt_p, cnt_p, b2)


def kernel(x, edge_index, edge_attr, Wl0, bl0, Wr0, Wl1, bl1, Wr1, Wl2, bl2, Wr2):
    src = edge_index[0]
    dst = edge_index[1]
    src_st = jnp.concatenate([src, src + N])

    w0 = jnp.concatenate([Wl0.T, Wr0.T], axis=1)
    w1 = jnp.concatenate([Wl1.T, Wr1.T], axis=1)
    w2 = jnp.concatenate([Wl2.T, Wr2.T], axis=1)
    b0 = bl0.reshape(1, D)
    b1 = bl1.reshape(1, D)
    b2 = bl2.reshape(1, D)

    _sc_seg_sum, _sc_counts = _build_sc_kernels()
    cnt_p = _sc_counts(dst)

    z0 = _tc_entry(x, w0)
    s0 = _sc_seg_sum(z0, src_st, dst)
    z1 = _tc_mid(s0, z0, cnt_p, b0, w1)
    s1 = _sc_seg_sum(z1, src_st, dst)
    z2 = _tc_mid(s1, z1, cnt_p, b1, w2)
    s2 = _sc_seg_sum(z2, src_st, dst)
    return _tc_final(s2, z2, cnt_p, b2)


# speedup vs baseline: 6.3072x; 1.1919x over previous
import functools

import jax
import jax.numpy as jnp
from jax import lax
from jax.experimental import pallas as pl
from jax.experimental.pallas import tpu as pltpu
from jax.experimental.pallas import tpu_sc as plsc

N = 10000
E = 160000
D = 256
H = 128
NC = 2
NS = 16
CH = 128
EP = E // NS
NF = EP // CH
TAIL = EP - NF * CH
ZR = 40
NRCH = N // ZR
CCH = 128
CEP = E // (NC * NS)
CNF = CEP // CCH
CTAIL = CEP - CNF * CCH

@functools.lru_cache(maxsize=None)
def _build_sc_kernels():
    mesh = plsc.VectorSubcoreMesh(
        core_axis_name="c", subcore_axis_name="s",
        num_cores=NC, num_subcores=NS,
    )
    seg = functools.partial(
        pl.kernel,
        out_type=jax.ShapeDtypeStruct((2 * N, H), jnp.float32),
        mesh=mesh,
        scratch_types=[
            pltpu.VMEM((EP,), jnp.int32),
            pltpu.VMEM((CH,), jnp.int32),
            pltpu.VMEM((CH,), jnp.int32),
            pltpu.VMEM((TAIL,), jnp.int32),
            pltpu.VMEM((CH, H), jnp.float32),
            pltpu.VMEM((CH, H), jnp.float32),
            pltpu.VMEM((TAIL, H), jnp.float32),
            pltpu.VMEM_SHARED((N, H), jnp.float32),
            pltpu.SemaphoreType.DMA,
            pltpu.SemaphoreType.DMA,
            pltpu.SemaphoreType.DMA,
            pltpu.SemaphoreType.DMA,
            pltpu.SemaphoreType.DMA,
            pltpu.SemaphoreType.DMA,
        ],
    )(_sc_seg_sum_body)
    cnts = functools.partial(
        pl.kernel,
        out_type=jax.ShapeDtypeStruct((NC * N, H), jnp.float32),
        mesh=mesh,
        scratch_types=[
            pltpu.VMEM((CCH,), jnp.int32),
            pltpu.VMEM((CCH,), jnp.int32),
            pltpu.VMEM((CTAIL,), jnp.int32),
            pltpu.VMEM((CCH, H), jnp.float32),
            pltpu.VMEM((ZR, H), jnp.float32),
            pltpu.VMEM_SHARED((N, H), jnp.float32),
            pltpu.SemaphoreType.DMA,
            pltpu.SemaphoreType.DMA,
            pltpu.SemaphoreType.DMA,
            pltpu.SemaphoreType.DMA,
            pltpu.SemaphoreType.DMA,
        ],
    )(_sc_counts_body)
    return seg, cnts


def _sc_seg_sum_body(hl_hbm, src_hbm, dst_hbm, out_hbm,
                     src_a, dst0, dst1, dst_t, rows0, rows1, rows_t, acc,
                     gsem, dsem0, dsem1, ssem0, ssem1, wsem):
    c = lax.axis_index("c")
    s = lax.axis_index("s")
    rows = (rows0, rows1)
    dsts = (dst0, dst1)
    dsems = (dsem0, dsem1)
    ssems = (ssem0, ssem1)

    pltpu.sync_copy(src_hbm.at[pl.ds(c * E + s * EP, EP)], src_a)

    def zbody(i, _):
        r = i // (H // 16)
        k = i % (H // 16)
        rows0[r, pl.ds(k * 16, 16)] = jnp.zeros((16,), jnp.float32)
        return 0
    lax.fori_loop(0, CH * (H // 16), zbody, 0)

    rlo = (s * NRCH) // NS
    rhi = ((s + 1) * NRCH) // NS
    zsrc = rows0.at[pl.ds(0, ZR)]

    def zcopy(j, _):
        pltpu.async_copy(zsrc, acc.at[pl.ds(j * ZR, ZR)], wsem)
        return 0
    lax.fori_loop(rlo, rhi, zcopy, 0)

    def zwait(j, _):
        pltpu.make_async_copy(zsrc, acc.at[pl.ds(j * ZR, ZR)], wsem).wait()
        return 0
    lax.fori_loop(rlo, rhi, zwait, 0)
    plsc.subcore_barrier()

    def src_sl(j):
        return src_a.at[pl.ds(j * CH, CH)]

    def dst_sl(j):
        return dst_hbm.at[pl.ds(s * EP + j * CH, CH)]

    pltpu.async_copy(dst_sl(0), dsts[0], dsems[0])
    pltpu.async_copy(hl_hbm.at[src_sl(0)], rows[0], gsem)

    def step(j, b, first, prefetch):
        pltpu.make_async_copy(hl_hbm.at[src_sl(j)], rows[b], gsem).wait()
        pltpu.make_async_copy(dst_sl(j), dsts[b], dsems[b]).wait()
        pltpu.async_copy(rows[b], acc.at[dsts[b]], ssems[b], add=True)
        if not first:
            pltpu.make_async_copy(
                rows[1 - b], acc.at[dsts[1 - b]], ssems[1 - b]).wait()

        if prefetch:
            @pl.when(j + 1 < NF)
            def _():
                pltpu.async_copy(dst_sl(j + 1), dsts[1 - b], dsems[1 - b])
                pltpu.async_copy(hl_hbm.at[src_sl(j + 1)], rows[1 - b], gsem)

    step(0, 0, first=True, prefetch=True)

    def body(i, _):
        step(1 + 2 * i, 1, first=False, prefetch=True)
        step(2 + 2 * i, 0, first=False, prefetch=True)
        return 0
    lax.fori_loop(0, (NF - 2) // 2, body, 0)

    step(NF - 1, 1, first=False, prefetch=False)

    toff = s * EP + NF * CH
    pltpu.async_copy(dst_hbm.at[pl.ds(toff, TAIL)], dst_t, dsems[0])
    pltpu.async_copy(hl_hbm.at[src_a.at[pl.ds(NF * CH, TAIL)]], rows_t, gsem)
    pltpu.make_async_copy(
        hl_hbm.at[src_a.at[pl.ds(NF * CH, TAIL)]], rows_t, gsem).wait()
    pltpu.make_async_copy(dst_hbm.at[pl.ds(toff, TAIL)], dst_t, dsems[0]).wait()
    pltpu.sync_copy(rows_t, acc.at[dst_t], add=True)

    pltpu.make_async_copy(rows[1], acc.at[dsts[1]], ssems[1]).wait()

    plsc.subcore_barrier()

    def wcopy(j, _):
        pltpu.async_copy(acc.at[pl.ds(j * ZR, ZR)],
                         out_hbm.at[pl.ds(c * N + j * ZR, ZR)], wsem)
        return 0
    lax.fori_loop(rlo, rhi, wcopy, 0)

    def wwait(j, _):
        pltpu.make_async_copy(acc.at[pl.ds(j * ZR, ZR)],
                              out_hbm.at[pl.ds(c * N + j * ZR, ZR)],
                              wsem).wait()
        return 0
    lax.fori_loop(rlo, rhi, wwait, 0)


def _sc_counts_body(dst_hbm, out_hbm, dst0, dst1, dst_t, ones_v, zero_v,
                    cacc, dsem0, dsem1, ssem0, ssem1, wsem):
    c = lax.axis_index("c")
    s = lax.axis_index("s")
    dsts = (dst0, dst1)
    dsems = (dsem0, dsem1)
    ssems = (ssem0, ssem1)

    def zbody(i, _):
        r = i // (H // 16)
        k = i % (H // 16)
        zero_v[r, pl.ds(k * 16, 16)] = jnp.zeros((16,), jnp.float32)
        return 0
    lax.fori_loop(0, ZR * (H // 16), zbody, 0)

    def obody(i, _):
        r = i // (H // 16)
        k = i % (H // 16)
        ones_v[r, pl.ds(k * 16, 16)] = jnp.ones((16,), jnp.float32)
        return 0
    lax.fori_loop(0, CCH * (H // 16), obody, 0)

    rlo = (s * NRCH) // NS
    rhi = ((s + 1) * NRCH) // NS

    def zcopy(j, _):
        pltpu.async_copy(zero_v, cacc.at[pl.ds(j * ZR, ZR)], wsem)
        return 0
    lax.fori_loop(rlo, rhi, zcopy, 0)

    def zwait(j, _):
        pltpu.make_async_copy(zero_v, cacc.at[pl.ds(j * ZR, ZR)], wsem).wait()
        return 0
    lax.fori_loop(rlo, rhi, zwait, 0)
    plsc.subcore_barrier()

    ebase = (c * NS + s) * CEP

    def dst_sl(j):
        return dst_hbm.at[pl.ds(ebase + j * CCH, CCH)]

    pltpu.async_copy(dst_sl(0), dsts[0], dsems[0])

    def step(j, b, first, prefetch):
        pltpu.make_async_copy(dst_sl(j), dsts[b], dsems[b]).wait()
        pltpu.async_copy(ones_v, cacc.at[dsts[b]], ssems[b], add=True)
        if not first:
            pltpu.make_async_copy(
                ones_v, cacc.at[dsts[1 - b]], ssems[1 - b]).wait()

        if prefetch:
            @pl.when(j + 1 < CNF)
            def _():
                pltpu.async_copy(dst_sl(j + 1), dsts[1 - b], dsems[1 - b])

    step(0, 0, first=True, prefetch=True)

    def body(i, _):
        step(1 + 2 * i, 1, first=False, prefetch=True)
        step(2 + 2 * i, 0, first=False, prefetch=True)
        return 0
    lax.fori_loop(0, (CNF - 3) // 2, body, 0)
    step(CNF - 2, (CNF - 2) % 2, first=False, prefetch=True)
    step(CNF - 1, (CNF - 1) % 2, first=False, prefetch=False)

    toff = ebase + CNF * CCH
    pltpu.async_copy(dst_hbm.at[pl.ds(toff, CTAIL)], dst_t, dsems[0])
    pltpu.make_async_copy(dst_hbm.at[pl.ds(toff, CTAIL)], dst_t,
                          dsems[0]).wait()
    pltpu.sync_copy(ones_v.at[pl.ds(0, CTAIL)], cacc.at[dst_t], add=True)

    lastb = (CNF - 1) % 2
    pltpu.make_async_copy(ones_v, cacc.at[dsts[lastb]], ssems[lastb]).wait()

    plsc.subcore_barrier()

    def wcopy(j, _):
        pltpu.async_copy(cacc.at[pl.ds(j * ZR, ZR)],
                         out_hbm.at[pl.ds(c * N + j * ZR, ZR)], wsem)
        return 0
    lax.fori_loop(rlo, rhi, wcopy, 0)

    def wwait(j, _):
        pltpu.make_async_copy(cacc.at[pl.ds(j * ZR, ZR)],
                              out_hbm.at[pl.ds(c * N + j * ZR, ZR)],
                              wsem).wait()
        return 0
    lax.fori_loop(rlo, rhi, wwait, 0)


R = 1000
NB = N // R
KSPLIT = 4


def _tc_entry_body(x_ref, w_ref, z_ref):
    z_ref[...] = jnp.dot(x_ref[...], w_ref[...],
                         preferred_element_type=jnp.float32)


def _tc_entry(x, wall):
    return pl.pallas_call(
        _tc_entry_body,
        grid=(NB, KSPLIT),
        in_specs=[
            pl.BlockSpec((R, D), lambda i, k: (i, 0)),
            pl.BlockSpec((D, H), lambda i, k: (0, k)),
        ],
        out_specs=pl.BlockSpec((R, H), lambda i, k: (k * (N // R) + i, 0)),
        out_shape=jax.ShapeDtypeStruct((4 * N, H), jnp.float32),
    )(x, wall)


def _tc_mid_body(sA_ref, sB_ref, hA_ref, hB_ref, cA_ref, cB_ref, b_ref,
                 w_ref, z_ref):
    cnt = cA_ref[:, 0:1] + cB_ref[:, 0:1]
    inv = 1.0 / jnp.maximum(cnt, 1.0)
    b = b_ref[...]
    h0 = sA_ref[...] * inv + hA_ref[...] + b[:, :H]
    h1 = sB_ref[...] * inv + hB_ref[...] + b[:, H:]
    h = jax.nn.relu(jnp.concatenate([h0, h1], axis=1))
    z_ref[...] = jnp.dot(h, w_ref[...], preferred_element_type=jnp.float32)


def _tc_mid(s_st, z_prev, cnt_p, b2, wall):
    nbr = N // R
    return pl.pallas_call(
        _tc_mid_body,
        grid=(NB, KSPLIT),
        in_specs=[
            pl.BlockSpec((R, H), lambda i, k: (i, 0)),
            pl.BlockSpec((R, H), lambda i, k: (nbr + i, 0)),
            pl.BlockSpec((R, H), lambda i, k: (2 * nbr + i, 0)),
            pl.BlockSpec((R, H), lambda i, k: (3 * nbr + i, 0)),
            pl.BlockSpec((R, H), lambda i, k: (i, 0)),
            pl.BlockSpec((R, H), lambda i, k: (nbr + i, 0)),
            pl.BlockSpec((1, D), lambda i, k: (0, 0)),
            pl.BlockSpec((D, H), lambda i, k: (0, k)),
        ],
        out_specs=pl.BlockSpec((R, H), lambda i, k: (k * nbr + i, 0)),
        out_shape=jax.ShapeDtypeStruct((4 * N, H), jnp.float32),
    )(s_st, s_st, z_prev, z_prev, cnt_p, cnt_p, b2, wall)


def _tc_final_body(sA_ref, sB_ref, hA_ref, hB_ref, cA_ref, cB_ref, b_ref,
                   o_ref):
    cnt = cA_ref[:, 0:1] + cB_ref[:, 0:1]
    inv = 1.0 / jnp.maximum(cnt, 1.0)
    b = b_ref[...]
    o_ref[:, :H] = sA_ref[...] * inv + hA_ref[...] + b[:, :H]
    o_ref[:, H:] = sB_ref[...] * inv + hB_ref[...] + b[:, H:]


def _tc_final(s_st, z_prev, cnt_p, b2):
    nbr = N // R
    return pl.pallas_call(
        _tc_final_body,
        grid=(NB,),
        in_specs=[
            pl.BlockSpec((R, H), lambda i: (i, 0)),
            pl.BlockSpec((R, H), lambda i: (nbr + i, 0)),
            pl.BlockSpec((R, H), lambda i: (2 * nbr + i, 0)),
            pl.BlockSpec((R, H), lambda i: (3 * nbr + i, 0)),
            pl.BlockSpec((R, H), lambda i: (i, 0)),
            pl.BlockSpec((R, H), lambda i: (nbr + i, 0)),
            pl.BlockSpec((1, D), lambda i: (0, 0)),
        ],
        out_specs=pl.BlockSpec((R, D), lambda i: (i, 0)),
        out_shape=jax.ShapeDtypeStruct((N, D), jnp.float32),
    )(s_st, s_st, z_prev, z_prev, cnt_p, cnt_p, b2)


def kernel(x, edge_index, edge_attr, Wl0, bl0, Wr0, Wl1, bl1, Wr1, Wl2, bl2, Wr2):
    src = edge_index[0]
    dst = edge_index[1]
    src_st = jnp.concatenate([src, src + N])

    w0 = jnp.concatenate([Wl0.T, Wr0.T], axis=1)
    w1 = jnp.concatenate([Wl1.T, Wr1.T], axis=1)
    w2 = jnp.concatenate([Wl2.T, Wr2.T], axis=1)
    b0 = bl0.reshape(1, D)
    b1 = bl1.reshape(1, D)
    b2 = bl2.reshape(1, D)

    _sc_seg_sum, _sc_counts = _build_sc_kernels()
    cnt_p = _sc_counts(dst)

    z0 = _tc_entry(x, w0)
    s0 = _sc_seg_sum(z0, src_st, dst)
    z1 = _tc_mid(s0, z0, cnt_p, b0, w1)
    s1 = _sc_seg_sum(z1, src_st, dst)
    z2 = _tc_mid(s1, z1, cnt_p, b1, w2)
    s2 = _sc_seg_sum(z2, src_st, dst)
    return _tc_final(s2, z2, cnt_p, b2)


# speedup vs baseline: 6.3406x; 1.0053x over previous
import functools

import jax
import jax.numpy as jnp
from jax import lax
from jax.experimental import pallas as pl
from jax.experimental.pallas import tpu as pltpu
from jax.experimental.pallas import tpu_sc as plsc

N = 10000
E = 160000
D = 256
H = 128
NC = 2
NS = 16
CH = 128
EP = E // NS
NF = EP // CH
TAIL = EP - NF * CH
ZR = 80
NRCH = N // ZR
CW = 8
CCH = 128
CEP = E // (NC * NS)
CNF = CEP // CCH
CTAIL = CEP - CNF * CCH

@functools.lru_cache(maxsize=None)
def _build_sc_kernels():
    mesh = plsc.VectorSubcoreMesh(
        core_axis_name="c", subcore_axis_name="s",
        num_cores=NC, num_subcores=NS,
    )
    seg = functools.partial(
        pl.kernel,
        out_type=jax.ShapeDtypeStruct((2 * N, H), jnp.float32),
        mesh=mesh,
        scratch_types=[
            pltpu.VMEM((EP,), jnp.int32),
            pltpu.VMEM((CH,), jnp.int32),
            pltpu.VMEM((CH,), jnp.int32),
            pltpu.VMEM((TAIL,), jnp.int32),
            pltpu.VMEM((CH, H), jnp.float32),
            pltpu.VMEM((CH, H), jnp.float32),
            pltpu.VMEM((TAIL, H), jnp.float32),
            pltpu.VMEM_SHARED((N, H), jnp.float32),
            pltpu.SemaphoreType.DMA,
            pltpu.SemaphoreType.DMA,
            pltpu.SemaphoreType.DMA,
            pltpu.SemaphoreType.DMA,
            pltpu.SemaphoreType.DMA,
            pltpu.SemaphoreType.DMA,
        ],
    )(_sc_seg_sum_body)
    cnts = functools.partial(
        pl.kernel,
        out_type=jax.ShapeDtypeStruct((NC * N, H), jnp.float32),
        mesh=mesh,
        scratch_types=[
            pltpu.VMEM((CCH,), jnp.int32),
            pltpu.VMEM((CCH,), jnp.int32),
            pltpu.VMEM((CTAIL,), jnp.int32),
            pltpu.VMEM((CCH, H), jnp.float32),
            pltpu.VMEM((ZR, H), jnp.float32),
            pltpu.VMEM_SHARED((N, H), jnp.float32),
            pltpu.SemaphoreType.DMA,
            pltpu.SemaphoreType.DMA,
            pltpu.SemaphoreType.DMA,
            pltpu.SemaphoreType.DMA,
            pltpu.SemaphoreType.DMA,
        ],
    )(_sc_counts_body)
    return seg, cnts


def _sc_seg_sum_body(hl_hbm, src_hbm, dst_hbm, out_hbm,
                     src_a, dst0, dst1, dst_t, rows0, rows1, rows_t, acc,
                     gsem, dsem0, dsem1, ssem0, ssem1, wsem):
    c = lax.axis_index("c")
    s = lax.axis_index("s")
    rows = (rows0, rows1)
    dsts = (dst0, dst1)
    dsems = (dsem0, dsem1)
    ssems = (ssem0, ssem1)

    pltpu.async_copy(src_hbm.at[pl.ds(c * E + s * EP, EP)], src_a, gsem)

    def zbody(i, _):
        r = i // (H // 16)
        k = i % (H // 16)
        rows0[r, pl.ds(k * 16, 16)] = jnp.zeros((16,), jnp.float32)
        return 0
    lax.fori_loop(0, CH * (H // 16), zbody, 0)

    rlo = (s * NRCH) // NS
    rhi = ((s + 1) * NRCH) // NS
    zsrc = rows0.at[pl.ds(0, ZR)]

    def zcopy(j, _):
        pltpu.async_copy(zsrc, acc.at[pl.ds(j * ZR, ZR)], wsem)
        return 0
    lax.fori_loop(rlo, rhi, zcopy, 0)

    def zwait(j, _):
        pltpu.make_async_copy(zsrc, acc.at[pl.ds(j * ZR, ZR)], wsem).wait()
        return 0
    lax.fori_loop(rlo, rhi, zwait, 0)
    pltpu.make_async_copy(
        src_hbm.at[pl.ds(c * E + s * EP, EP)], src_a, gsem).wait()
    plsc.subcore_barrier()

    def src_sl(j):
        return src_a.at[pl.ds(j * CH, CH)]

    def dst_sl(j):
        return dst_hbm.at[pl.ds(s * EP + j * CH, CH)]

    pltpu.async_copy(dst_sl(0), dsts[0], dsems[0])
    pltpu.async_copy(hl_hbm.at[src_sl(0)], rows[0], gsem)

    def step(j, b, first, prefetch):
        pltpu.make_async_copy(hl_hbm.at[src_sl(j)], rows[b], gsem).wait()
        pltpu.make_async_copy(dst_sl(j), dsts[b], dsems[b]).wait()
        pltpu.async_copy(rows[b], acc.at[dsts[b]], ssems[b], add=True)
        if not first:
            pltpu.make_async_copy(
                rows[1 - b], acc.at[dsts[1 - b]], ssems[1 - b]).wait()

        if prefetch:
            @pl.when(j + 1 < NF)
            def _():
                pltpu.async_copy(dst_sl(j + 1), dsts[1 - b], dsems[1 - b])
                pltpu.async_copy(hl_hbm.at[src_sl(j + 1)], rows[1 - b], gsem)

    step(0, 0, first=True, prefetch=True)

    def body(i, _):
        step(1 + 2 * i, 1, first=False, prefetch=True)
        step(2 + 2 * i, 0, first=False, prefetch=True)
        return 0
    lax.fori_loop(0, (NF - 2) // 2, body, 0)

    step(NF - 1, 1, first=False, prefetch=False)

    toff = s * EP + NF * CH
    pltpu.async_copy(dst_hbm.at[pl.ds(toff, TAIL)], dst_t, dsems[0])
    pltpu.async_copy(hl_hbm.at[src_a.at[pl.ds(NF * CH, TAIL)]], rows_t, gsem)
    pltpu.make_async_copy(
        hl_hbm.at[src_a.at[pl.ds(NF * CH, TAIL)]], rows_t, gsem).wait()
    pltpu.make_async_copy(dst_hbm.at[pl.ds(toff, TAIL)], dst_t, dsems[0]).wait()
    pltpu.sync_copy(rows_t, acc.at[dst_t], add=True)

    pltpu.make_async_copy(rows[1], acc.at[dsts[1]], ssems[1]).wait()

    plsc.subcore_barrier()

    def wcopy(j, _):
        pltpu.async_copy(acc.at[pl.ds(j * ZR, ZR)],
                         out_hbm.at[pl.ds(c * N + j * ZR, ZR)], wsem)
        return 0
    lax.fori_loop(rlo, rhi, wcopy, 0)

    def wwait(j, _):
        pltpu.make_async_copy(acc.at[pl.ds(j * ZR, ZR)],
                              out_hbm.at[pl.ds(c * N + j * ZR, ZR)],
                              wsem).wait()
        return 0
    lax.fori_loop(rlo, rhi, wwait, 0)


def _sc_counts_body(dst_hbm, out_hbm, dst0, dst1, dst_t, ones_v, zero_v,
                    cacc, dsem0, dsem1, ssem0, ssem1, wsem):
    c = lax.axis_index("c")
    s = lax.axis_index("s")
    dsts = (dst0, dst1)
    dsems = (dsem0, dsem1)
    ssems = (ssem0, ssem1)

    def zbody(i, _):
        r = i // (H // 16)
        k = i % (H // 16)
        zero_v[r, pl.ds(k * 16, 16)] = jnp.zeros((16,), jnp.float32)
        return 0
    lax.fori_loop(0, ZR * (H // 16), zbody, 0)

    def obody(i, _):
        r = i // (H // 16)
        k = i % (H // 16)
        ones_v[r, pl.ds(k * 16, 16)] = jnp.ones((16,), jnp.float32)
        return 0
    lax.fori_loop(0, CCH * (H // 16), obody, 0)

    rlo = (s * NRCH) // NS
    rhi = ((s + 1) * NRCH) // NS

    def zcopy(j, _):
        pltpu.async_copy(zero_v, cacc.at[pl.ds(j * ZR, ZR)], wsem)
        return 0
    lax.fori_loop(rlo, rhi, zcopy, 0)

    def zwait(j, _):
        pltpu.make_async_copy(zero_v, cacc.at[pl.ds(j * ZR, ZR)], wsem).wait()
        return 0
    lax.fori_loop(rlo, rhi, zwait, 0)
    plsc.subcore_barrier()

    ebase = (c * NS + s) * CEP

    def dst_sl(j):
        return dst_hbm.at[pl.ds(ebase + j * CCH, CCH)]

    pltpu.async_copy(dst_sl(0), dsts[0], dsems[0])

    def step(j, b, first, prefetch):
        pltpu.make_async_copy(dst_sl(j), dsts[b], dsems[b]).wait()
        pltpu.async_copy(ones_v, cacc.at[dsts[b]], ssems[b], add=True)
        if not first:
            pltpu.make_async_copy(
                ones_v, cacc.at[dsts[1 - b]], ssems[1 - b]).wait()

        if prefetch:
            @pl.when(j + 1 < CNF)
            def _():
                pltpu.async_copy(dst_sl(j + 1), dsts[1 - b], dsems[1 - b])

    step(0, 0, first=True, prefetch=True)

    def body(i, _):
        step(1 + 2 * i, 1, first=False, prefetch=True)
        step(2 + 2 * i, 0, first=False, prefetch=True)
        return 0
    lax.fori_loop(0, (CNF - 3) // 2, body, 0)
    step(CNF - 2, (CNF - 2) % 2, first=False, prefetch=True)
    step(CNF - 1, (CNF - 1) % 2, first=False, prefetch=False)

    toff = ebase + CNF * CCH
    pltpu.async_copy(dst_hbm.at[pl.ds(toff, CTAIL)], dst_t, dsems[0])
    pltpu.make_async_copy(dst_hbm.at[pl.ds(toff, CTAIL)], dst_t,
                          dsems[0]).wait()
    pltpu.sync_copy(ones_v.at[pl.ds(0, CTAIL)], cacc.at[dst_t], add=True)

    lastb = (CNF - 1) % 2
    pltpu.make_async_copy(ones_v, cacc.at[dsts[lastb]], ssems[lastb]).wait()

    plsc.subcore_barrier()

    def wcopy(j, _):
        pltpu.async_copy(cacc.at[pl.ds(j * ZR, ZR)],
                         out_hbm.at[pl.ds(c * N + j * ZR, ZR)], wsem)
        return 0
    lax.fori_loop(rlo, rhi, wcopy, 0)

    def wwait(j, _):
        pltpu.make_async_copy(cacc.at[pl.ds(j * ZR, ZR)],
                              out_hbm.at[pl.ds(c * N + j * ZR, ZR)],
                              wsem).wait()
        return 0
    lax.fori_loop(rlo, rhi, wwait, 0)


R = 1000
NB = N // R
KSPLIT = 4


def _tc_entry_body(x_ref, w_ref, z_ref):
    z_ref[...] = jnp.dot(x_ref[...], w_ref[...],
                         preferred_element_type=jnp.float32)


def _tc_entry(x, wall):
    return pl.pallas_call(
        _tc_entry_body,
        grid=(NB, KSPLIT),
        in_specs=[
            pl.BlockSpec((R, D), lambda i, k: (i, 0)),
            pl.BlockSpec((D, H), lambda i, k: (0, k)),
        ],
        out_specs=pl.BlockSpec((R, H), lambda i, k: (k * (N // R) + i, 0)),
        out_shape=jax.ShapeDtypeStruct((4 * N, H), jnp.float32),
    )(x, wall)


def _tc_mid_body(sA_ref, sB_ref, hA_ref, hB_ref, cA_ref, cB_ref, b_ref,
                 w_ref, z_ref):
    cnt = cA_ref[:, 0:1] + cB_ref[:, 0:1]
    inv = 1.0 / jnp.maximum(cnt, 1.0)
    b = b_ref[...]
    h0 = sA_ref[...] * inv + hA_ref[...] + b[:, :H]
    h1 = sB_ref[...] * inv + hB_ref[...] + b[:, H:]
    h = jax.nn.relu(jnp.concatenate([h0, h1], axis=1))
    z_ref[...] = jnp.dot(h, w_ref[...], preferred_element_type=jnp.float32)


def _tc_mid(s_st, z_prev, cnt_p, b2, wall):
    nbr = N // R
    return pl.pallas_call(
        _tc_mid_body,
        grid=(NB, KSPLIT),
        in_specs=[
            pl.BlockSpec((R, H), lambda i, k: (i, 0)),
            pl.BlockSpec((R, H), lambda i, k: (nbr + i, 0)),
            pl.BlockSpec((R, H), lambda i, k: (2 * nbr + i, 0)),
            pl.BlockSpec((R, H), lambda i, k: (3 * nbr + i, 0)),
            pl.BlockSpec((R, H), lambda i, k: (i, 0)),
            pl.BlockSpec((R, H), lambda i, k: (nbr + i, 0)),
            pl.BlockSpec((1, D), lambda i, k: (0, 0)),
            pl.BlockSpec((D, H), lambda i, k: (0, k)),
        ],
        out_specs=pl.BlockSpec((R, H), lambda i, k: (k * nbr + i, 0)),
        out_shape=jax.ShapeDtypeStruct((4 * N, H), jnp.float32),
    )(s_st, s_st, z_prev, z_prev, cnt_p, cnt_p, b2, wall)


def _tc_final_body(sA_ref, sB_ref, hA_ref, hB_ref, cA_ref, cB_ref, b_ref,
                   o_ref):
    cnt = cA_ref[:, 0:1] + cB_ref[:, 0:1]
    inv = 1.0 / jnp.maximum(cnt, 1.0)
    b = b_ref[...]
    o_ref[:, :H] = sA_ref[...] * inv + hA_ref[...] + b[:, :H]
    o_ref[:, H:] = sB_ref[...] * inv + hB_ref[...] + b[:, H:]


def _tc_final(s_st, z_prev, cnt_p, b2):
    nbr = N // R
    return pl.pallas_call(
        _tc_final_body,
        grid=(NB,),
        in_specs=[
            pl.BlockSpec((R, H), lambda i: (i, 0)),
            pl.BlockSpec((R, H), lambda i: (nbr + i, 0)),
            pl.BlockSpec((R, H), lambda i: (2 * nbr + i, 0)),
            pl.BlockSpec((R, H), lambda i: (3 * nbr + i, 0)),
            pl.BlockSpec((R, H), lambda i: (i, 0)),
            pl.BlockSpec((R, H), lambda i: (nbr + i, 0)),
            pl.BlockSpec((1, D), lambda i: (0, 0)),
        ],
        out_specs=pl.BlockSpec((R, D), lambda i: (i, 0)),
        out_shape=jax.ShapeDtypeStruct((N, D), jnp.float32),
    )(s_st, s_st, z_prev, z_prev, cnt_p, cnt_p, b2)


def kernel(x, edge_index, edge_attr, Wl0, bl0, Wr0, Wl1, bl1, Wr1, Wl2, bl2, Wr2):
    src = edge_index[0]
    dst = edge_index[1]
    src_st = jnp.concatenate([src, src + N])

    w0 = jnp.concatenate([Wl0.T, Wr0.T], axis=1)
    w1 = jnp.concatenate([Wl1.T, Wr1.T], axis=1)
    w2 = jnp.concatenate([Wl2.T, Wr2.T], axis=1)
    b0 = bl0.reshape(1, D)
    b1 = bl1.reshape(1, D)
    b2 = bl2.reshape(1, D)

    _sc_seg_sum, _sc_counts = _build_sc_kernels()
    cnt_p = _sc_counts(dst)

    z0 = _tc_entry(x, w0)
    s0 = _sc_seg_sum(z0, src_st, dst)
    z1 = _tc_mid(s0, z0, cnt_p, b0, w1)
    s1 = _sc_seg_sum(z1, src_st, dst)
    z2 = _tc_mid(s1, z1, cnt_p, b1, w2)
    s2 = _sc_seg_sum(z2, src_st, dst)
    return _tc_final(s2, z2, cnt_p, b2)


# speedup vs baseline: 6.4025x; 1.0098x over previous
import functools

import jax
import jax.numpy as jnp
from jax import lax
from jax.experimental import pallas as pl
from jax.experimental.pallas import tpu as pltpu
from jax.experimental.pallas import tpu_sc as plsc

N = 10000
E = 160000
D = 256
H = 128
NC = 2
NS = 16
CH = 128
EP = E // NS
NF = EP // CH
TAIL = EP - NF * CH
ZR = 80
NRCH = N // ZR
CW = 8
CCH = 128
CEP = E // (NC * NS)
CNF = CEP // CCH
CTAIL = CEP - CNF * CCH

@functools.lru_cache(maxsize=None)
def _build_sc_kernels():
    mesh = plsc.VectorSubcoreMesh(
        core_axis_name="c", subcore_axis_name="s",
        num_cores=NC, num_subcores=NS,
    )
    seg = functools.partial(
        pl.kernel,
        out_type=jax.ShapeDtypeStruct((2 * N, H), jnp.float32),
        mesh=mesh,
        scratch_types=[
            pltpu.VMEM((EP,), jnp.int32),
            pltpu.VMEM((CH,), jnp.int32),
            pltpu.VMEM((CH,), jnp.int32),
            pltpu.VMEM((TAIL,), jnp.int32),
            pltpu.VMEM((CH, H), jnp.float32),
            pltpu.VMEM((CH, H), jnp.float32),
            pltpu.VMEM((TAIL, H), jnp.float32),
            pltpu.VMEM_SHARED((N, H), jnp.float32),
            pltpu.SemaphoreType.DMA,
            pltpu.SemaphoreType.DMA,
            pltpu.SemaphoreType.DMA,
            pltpu.SemaphoreType.DMA,
            pltpu.SemaphoreType.DMA,
            pltpu.SemaphoreType.DMA,
        ],
    )(_sc_seg_sum_body)
    cnts = functools.partial(
        pl.kernel,
        out_type=jax.ShapeDtypeStruct((NC * N, H), jnp.float32),
        mesh=mesh,
        scratch_types=[
            pltpu.VMEM((CCH,), jnp.int32),
            pltpu.VMEM((CCH,), jnp.int32),
            pltpu.VMEM((CTAIL,), jnp.int32),
            pltpu.VMEM((CCH, H), jnp.float32),
            pltpu.VMEM((ZR, H), jnp.float32),
            pltpu.VMEM_SHARED((N, H), jnp.float32),
            pltpu.SemaphoreType.DMA,
            pltpu.SemaphoreType.DMA,
            pltpu.SemaphoreType.DMA,
            pltpu.SemaphoreType.DMA,
            pltpu.SemaphoreType.DMA,
        ],
    )(_sc_counts_body)
    return seg, cnts


def _sc_seg_sum_body(hl_hbm, src_hbm, dst_hbm, out_hbm,
                     src_a, dst0, dst1, dst_t, rows0, rows1, rows_t, acc,
                     gsem, dsem0, dsem1, ssem0, ssem1, wsem):
    c = lax.axis_index("c")
    s = lax.axis_index("s")
    rows = (rows0, rows1)
    dsts = (dst0, dst1)
    dsems = (dsem0, dsem1)
    ssems = (ssem0, ssem1)

    pltpu.async_copy(src_hbm.at[pl.ds(c * E + s * EP, EP)], src_a, gsem)

    def zbody(i, _):
        r = i // (H // 16)
        k = i % (H // 16)
        rows0[r, pl.ds(k * 16, 16)] = jnp.zeros((16,), jnp.float32)
        return 0
    lax.fori_loop(0, CH * (H // 16), zbody, 0)

    rlo = (s * NRCH) // NS
    rhi = ((s + 1) * NRCH) // NS
    zsrc = rows0.at[pl.ds(0, ZR)]

    def zcopy(j, _):
        pltpu.async_copy(zsrc, acc.at[pl.ds(j * ZR, ZR)], wsem)
        return 0
    lax.fori_loop(rlo, rhi, zcopy, 0)

    def zwait(j, _):
        pltpu.make_async_copy(zsrc, acc.at[pl.ds(j * ZR, ZR)], wsem).wait()
        return 0
    lax.fori_loop(rlo, rhi, zwait, 0)
    pltpu.make_async_copy(
        src_hbm.at[pl.ds(c * E + s * EP, EP)], src_a, gsem).wait()
    plsc.subcore_barrier()

    def src_sl(j):
        return src_a.at[pl.ds(j * CH, CH)]

    def dst_sl(j):
        return dst_hbm.at[pl.ds(s * EP + j * CH, CH)]

    pltpu.async_copy(dst_sl(0), dsts[0], dsems[0])
    pltpu.async_copy(hl_hbm.at[src_sl(0)], rows[0], gsem)

    def step(j, b, first, prefetch):
        pltpu.make_async_copy(hl_hbm.at[src_sl(j)], rows[b], gsem).wait()
        pltpu.make_async_copy(dst_sl(j), dsts[b], dsems[b]).wait()
        pltpu.async_copy(rows[b], acc.at[dsts[b]], ssems[b], add=True)
        if not first:
            pltpu.make_async_copy(
                rows[1 - b], acc.at[dsts[1 - b]], ssems[1 - b]).wait()

        if prefetch:
            @pl.when(j + 1 < NF)
            def _():
                pltpu.async_copy(dst_sl(j + 1), dsts[1 - b], dsems[1 - b])
                pltpu.async_copy(hl_hbm.at[src_sl(j + 1)], rows[1 - b], gsem)

    step(0, 0, first=True, prefetch=True)

    def body(i, _):
        step(1 + 2 * i, 1, first=False, prefetch=True)
        step(2 + 2 * i, 0, first=False, prefetch=True)
        return 0
    lax.fori_loop(0, (NF - 2) // 2, body, 0)

    step(NF - 1, 1, first=False, prefetch=False)

    toff = s * EP + NF * CH
    pltpu.async_copy(dst_hbm.at[pl.ds(toff, TAIL)], dst_t, dsems[0])
    pltpu.async_copy(hl_hbm.at[src_a.at[pl.ds(NF * CH, TAIL)]], rows_t, gsem)
    pltpu.make_async_copy(
        hl_hbm.at[src_a.at[pl.ds(NF * CH, TAIL)]], rows_t, gsem).wait()
    pltpu.make_async_copy(dst_hbm.at[pl.ds(toff, TAIL)], dst_t, dsems[0]).wait()
    pltpu.sync_copy(rows_t, acc.at[dst_t], add=True)

    pltpu.make_async_copy(rows[1], acc.at[dsts[1]], ssems[1]).wait()

    plsc.subcore_barrier()

    def wcopy(j, _):
        pltpu.async_copy(acc.at[pl.ds(j * ZR, ZR)],
                         out_hbm.at[pl.ds(c * N + j * ZR, ZR)], wsem)
        return 0
    lax.fori_loop(rlo, rhi, wcopy, 0)

    def wwait(j, _):
        pltpu.make_async_copy(acc.at[pl.ds(j * ZR, ZR)],
                              out_hbm.at[pl.ds(c * N + j * ZR, ZR)],
                              wsem).wait()
        return 0
    lax.fori_loop(rlo, rhi, wwait, 0)


def _sc_counts_body(dst_hbm, out_hbm, dst0, dst1, dst_t, ones_v, zero_v,
                    cacc, dsem0, dsem1, ssem0, ssem1, wsem):
    c = lax.axis_index("c")
    s = lax.axis_index("s")
    dsts = (dst0, dst1)
    dsems = (dsem0, dsem1)
    ssems = (ssem0, ssem1)

    def zbody(i, _):
        r = i // (H // 16)
        k = i % (H // 16)
        zero_v[r, pl.ds(k * 16, 16)] = jnp.zeros((16,), jnp.float32)
        return 0
    lax.fori_loop(0, ZR * (H // 16), zbody, 0)

    def obody(i, _):
        r = i // (H // 16)
        k = i % (H // 16)
        ones_v[r, pl.ds(k * 16, 16)] = jnp.ones((16,), jnp.float32)
        return 0
    lax.fori_loop(0, CCH * (H // 16), obody, 0)

    rlo = (s * NRCH) // NS
    rhi = ((s + 1) * NRCH) // NS

    def zcopy(j, _):
        pltpu.async_copy(zero_v, cacc.at[pl.ds(j * ZR, ZR)], wsem)
        return 0
    lax.fori_loop(rlo, rhi, zcopy, 0)

    def zwait(j, _):
        pltpu.make_async_copy(zero_v, cacc.at[pl.ds(j * ZR, ZR)], wsem).wait()
        return 0
    lax.fori_loop(rlo, rhi, zwait, 0)
    plsc.subcore_barrier()

    ebase = (c * NS + s) * CEP

    def dst_sl(j):
        return dst_hbm.at[pl.ds(ebase + j * CCH, CCH)]

    pltpu.async_copy(dst_sl(0), dsts[0], dsems[0])

    def step(j, b, first, prefetch):
        pltpu.make_async_copy(dst_sl(j), dsts[b], dsems[b]).wait()
        pltpu.async_copy(ones_v, cacc.at[dsts[b]], ssems[b], add=True)
        if not first:
            pltpu.make_async_copy(
                ones_v, cacc.at[dsts[1 - b]], ssems[1 - b]).wait()

        if prefetch:
            @pl.when(j + 1 < CNF)
            def _():
                pltpu.async_copy(dst_sl(j + 1), dsts[1 - b], dsems[1 - b])

    step(0, 0, first=True, prefetch=True)

    def body(i, _):
        step(1 + 2 * i, 1, first=False, prefetch=True)
        step(2 + 2 * i, 0, first=False, prefetch=True)
        return 0
    lax.fori_loop(0, (CNF - 3) // 2, body, 0)
    step(CNF - 2, (CNF - 2) % 2, first=False, prefetch=True)
    step(CNF - 1, (CNF - 1) % 2, first=False, prefetch=False)

    toff = ebase + CNF * CCH
    pltpu.async_copy(dst_hbm.at[pl.ds(toff, CTAIL)], dst_t, dsems[0])
    pltpu.make_async_copy(dst_hbm.at[pl.ds(toff, CTAIL)], dst_t,
                          dsems[0]).wait()
    pltpu.sync_copy(ones_v.at[pl.ds(0, CTAIL)], cacc.at[dst_t], add=True)

    lastb = (CNF - 1) % 2
    pltpu.make_async_copy(ones_v, cacc.at[dsts[lastb]], ssems[lastb]).wait()

    plsc.subcore_barrier()

    def wcopy(j, _):
        pltpu.async_copy(cacc.at[pl.ds(j * ZR, ZR)],
                         out_hbm.at[pl.ds(c * N + j * ZR, ZR)], wsem)
        return 0
    lax.fori_loop(rlo, rhi, wcopy, 0)

    def wwait(j, _):
        pltpu.make_async_copy(cacc.at[pl.ds(j * ZR, ZR)],
                              out_hbm.at[pl.ds(c * N + j * ZR, ZR)],
                              wsem).wait()
        return 0
    lax.fori_loop(rlo, rhi, wwait, 0)


R = 1000
NB = N // R
KSPLIT = 4


def _tc_entry_body(x_ref, w_ref, cA_ref, cB_ref, z_ref, inv_ref):
    z_ref[...] = jnp.dot(x_ref[...], w_ref[...],
                         preferred_element_type=jnp.float32)
    cnt = cA_ref[:, :CW] + cB_ref[:, :CW]
    inv_ref[...] = 1.0 / jnp.maximum(cnt, 1.0)


def _tc_entry(x, wall, cnt_p):
    nbr = N // R
    return pl.pallas_call(
        _tc_entry_body,
        grid=(NB, KSPLIT),
        in_specs=[
            pl.BlockSpec((R, D), lambda i, k: (i, 0)),
            pl.BlockSpec((D, H), lambda i, k: (0, k)),
            pl.BlockSpec((R, H), lambda i, k: (i, 0)),
            pl.BlockSpec((R, H), lambda i, k: (nbr + i, 0)),
        ],
        out_specs=[
            pl.BlockSpec((R, H), lambda i, k: (k * nbr + i, 0)),
            pl.BlockSpec((R, CW), lambda i, k: (i, 0)),
        ],
        out_shape=[
            jax.ShapeDtypeStruct((4 * N, H), jnp.float32),
            jax.ShapeDtypeStruct((N, CW), jnp.float32),
        ],
    )(x, wall, cnt_p, cnt_p)


def _tc_mid_body(sA_ref, sB_ref, hA_ref, hB_ref, inv_ref, b_ref,
                 w_ref, z_ref):
    inv = inv_ref[:, 0:1]
    b = b_ref[...]
    h0 = sA_ref[...] * inv + hA_ref[...] + b[:, :H]
    h1 = sB_ref[...] * inv + hB_ref[...] + b[:, H:]
    h = jax.nn.relu(jnp.concatenate([h0, h1], axis=1))
    z_ref[...] = jnp.dot(h, w_ref[...], preferred_element_type=jnp.float32)


def _tc_mid(s_st, z_prev, invc, b2, wall):
    nbr = N // R
    return pl.pallas_call(
        _tc_mid_body,
        grid=(NB, KSPLIT),
        in_specs=[
            pl.BlockSpec((R, H), lambda i, k: (i, 0)),
            pl.BlockSpec((R, H), lambda i, k: (nbr + i, 0)),
            pl.BlockSpec((R, H), lambda i, k: (2 * nbr + i, 0)),
            pl.BlockSpec((R, H), lambda i, k: (3 * nbr + i, 0)),
            pl.BlockSpec((R, CW), lambda i, k: (i, 0)),
            pl.BlockSpec((1, D), lambda i, k: (0, 0)),
            pl.BlockSpec((D, H), lambda i, k: (0, k)),
        ],
        out_specs=pl.BlockSpec((R, H), lambda i, k: (k * nbr + i, 0)),
        out_shape=jax.ShapeDtypeStruct((4 * N, H), jnp.float32),
    )(s_st, s_st, z_prev, z_prev, invc, b2, wall)


def _tc_final_body(sA_ref, sB_ref, hA_ref, hB_ref, inv_ref, b_ref,
                   o_ref):
    inv = inv_ref[:, 0:1]
    b = b_ref[...]
    o_ref[:, :H] = sA_ref[...] * inv + hA_ref[...] + b[:, :H]
    o_ref[:, H:] = sB_ref[...] * inv + hB_ref[...] + b[:, H:]


def _tc_final(s_st, z_prev, invc, b2):
    nbr = N // R
    return pl.pallas_call(
        _tc_final_body,
        grid=(NB,),
        in_specs=[
            pl.BlockSpec((R, H), lambda i: (i, 0)),
            pl.BlockSpec((R, H), lambda i: (nbr + i, 0)),
            pl.BlockSpec((R, H), lambda i: (2 * nbr + i, 0)),
            pl.BlockSpec((R, H), lambda i: (3 * nbr + i, 0)),
            pl.BlockSpec((R, CW), lambda i: (i, 0)),
            pl.BlockSpec((1, D), lambda i: (0, 0)),
        ],
        out_specs=pl.BlockSpec((R, D), lambda i: (i, 0)),
        out_shape=jax.ShapeDtypeStruct((N, D), jnp.float32),
    )(s_st, s_st, z_prev, z_prev, invc, b2)


def kernel(x, edge_index, edge_attr, Wl0, bl0, Wr0, Wl1, bl1, Wr1, Wl2, bl2, Wr2):
    src = edge_index[0]
    dst = edge_index[1]
    src_st = jnp.concatenate([src, src + N])

    w0 = jnp.concatenate([Wl0.T, Wr0.T], axis=1)
    w1 = jnp.concatenate([Wl1.T, Wr1.T], axis=1)
    w2 = jnp.concatenate([Wl2.T, Wr2.T], axis=1)
    b0 = bl0.reshape(1, D)
    b1 = bl1.reshape(1, D)
    b2 = bl2.reshape(1, D)

    _sc_seg_sum, _sc_counts = _build_sc_kernels()
    cnt_p = _sc_counts(dst)

    z0, invc = _tc_entry(x, w0, cnt_p)
    s0 = _sc_seg_sum(z0, src_st, dst)
    z1 = _tc_mid(s0, z0, invc, b0, w1)
    s1 = _sc_seg_sum(z1, src_st, dst)
    z2 = _tc_mid(s1, z1, invc, b1, w2)
    s2 = _sc_seg_sum(z2, src_st, dst)
    return _tc_final(s2, z2, invc, b2)
